# Initial kernel scaffold; baseline (speedup 1.0000x reference)
import jax
import jax.numpy as jnp
from jax.experimental import pallas as pl


def kernel(x, W_r, b_r, W_s, gamma_s, beta_s, W_e, gamma_e, beta_e):
    raise NotImplementedError("write your pallas kernel here")


# speedup vs baseline: 1.3879x; 1.3879x over previous
import jax
import jax.numpy as jnp
from jax.experimental import pallas as pl
from jax.experimental.pallas import tpu as pltpu

B, C_IN, C_OUT, H, W = 64, 192, 192, 14, 14
NUM_EXPERTS, TOP_K = 8, 2
EPS = 1e-5
HW = H * W


def _moe_body(x_ref, wr_ref, br_ref, ws_ref, gs_ref, bs_ref,
              we_ref, ge_ref, be_ref, out_ref):
    xb = x_ref[0]
    pooled = jnp.mean(xb, axis=1, keepdims=True)
    logits = jnp.dot(wr_ref[...], pooled,
                     preferred_element_type=jnp.float32) + br_ref[...]
    m = jnp.max(logits)
    e = jnp.exp(logits - m)
    p = e / jnp.sum(e)
    iota = jax.lax.broadcasted_iota(jnp.int32, (NUM_EXPERTS, 1), 0)
    v0 = jnp.max(p)
    i0 = jnp.min(jnp.where(p == v0, iota, NUM_EXPERTS)).astype(jnp.int32)
    p1 = jnp.where(iota == i0, -jnp.inf, p)
    v1 = jnp.max(p1)
    i1 = jnp.min(jnp.where(p1 == v1, iota, NUM_EXPERTS)).astype(jnp.int32)
    denom = v0 + v1 + 1e-9
    w0 = v0 / denom
    w1 = v1 / denom

    rs = (1.0 + EPS) ** -0.5

    ys = jnp.dot(ws_ref[...], xb, preferred_element_type=jnp.float32)
    ys = ys * (gs_ref[...] * rs) + bs_ref[...]
    ys = ys * jax.nn.sigmoid(ys)

    def expert(i, wgt):
        y = jnp.dot(we_ref[i], xb, preferred_element_type=jnp.float32)
        y = y * (ge_ref[i] * rs) + be_ref[i]
        y = y * jax.nn.sigmoid(y)
        return y * wgt

    out_ref[0] = ys + expert(i0, w0) + expert(i1, w1)


def kernel(x, W_r, b_r, W_s, gamma_s, beta_s, W_e, gamma_e, beta_e):
    x2 = x.reshape(B, C_IN, HW)
    out = pl.pallas_call(
        _moe_body,
        grid=(B,),
        in_specs=[
            pl.BlockSpec((1, C_IN, HW), lambda b: (b, 0, 0)),
            pl.BlockSpec((NUM_EXPERTS, C_IN), lambda b: (0, 0)),
            pl.BlockSpec((NUM_EXPERTS, 1), lambda b: (0, 0)),
            pl.BlockSpec((C_OUT, C_IN), lambda b: (0, 0)),
            pl.BlockSpec((C_OUT, 1), lambda b: (0, 0)),
            pl.BlockSpec((C_OUT, 1), lambda b: (0, 0)),
            pl.BlockSpec((NUM_EXPERTS, C_OUT, C_IN), lambda b: (0, 0, 0)),
            pl.BlockSpec((NUM_EXPERTS, C_OUT, 1), lambda b: (0, 0, 0)),
            pl.BlockSpec((NUM_EXPERTS, C_OUT, 1), lambda b: (0, 0, 0)),
        ],
        out_specs=pl.BlockSpec((1, C_OUT, HW), lambda b: (b, 0, 0)),
        out_shape=jax.ShapeDtypeStruct((B, C_OUT, HW), jnp.float32),
        compiler_params=pltpu.CompilerParams(
            dimension_semantics=("arbitrary",)),
    )(x2, W_r, b_r.reshape(NUM_EXPERTS, 1), W_s,
      gamma_s.reshape(C_OUT, 1), beta_s.reshape(C_OUT, 1), W_e,
      gamma_e.reshape(NUM_EXPERTS, C_OUT, 1),
      beta_e.reshape(NUM_EXPERTS, C_OUT, 1))
    return out.reshape(B, C_OUT, H, W)
